# baseline (speedup 1.0000x reference)
import functools

import jax
import jax.numpy as jnp
from jax import lax
from jax.experimental import pallas as pl
from jax.experimental.pallas import tpu as pltpu
from jax.experimental.pallas import tpu_sc as plsc

_N = 16384
_C = 1000
_HC = 1024
_NW = 32
_RW = _N // _NW


def _sc_alpha_body(t2_hbm, ag_hbm, hist_hbm, tgth_v, tgt2_v, ones_v,
                   zero_v, htg_v, ag_v, hist_sh):
    c = lax.axis_index("c")
    s = lax.axis_index("s")
    w = s * 2 + c

    for i in range(8):
        ones_v[pl.ds(i * 16, 16)] = jnp.ones((16,), jnp.float32)
    for i in range(64):
        zero_v[pl.ds(i * 16, 16)] = jnp.zeros((16,), jnp.float32)

    pltpu.sync_copy(t2_hbm.at[w], tgt2_v)

    @pl.when(s == 0)
    def _():
        pltpu.sync_copy(zero_v, hist_sh)

    plsc.subcore_barrier()
    for k in range(2):
        pltpu.sync_copy(t2_hbm.at[2 * s + k], tgth_v)
        for q in range(4):
            pltpu.sync_copy(ones_v, hist_sh.at[tgth_v.at[q]], add=True)
    plsc.subcore_barrier()

    @pl.when(s == 0)
    def _():
        pltpu.sync_copy(hist_sh, hist_hbm.at[c])

    plsc.subcore_barrier()

    for q in range(4):
        pltpu.sync_copy(hist_sh.at[tgt2_v.at[q]],
                        htg_v.at[pl.ds(q * 128, 128)])
    inv_n = 1.0 / _N
    for j in range(_RW // 16):
        ht16 = htg_v[pl.ds(j * 16, 16)]
        a16 = 1.0 / (ht16 * inv_n + 1e-5)
        ag_v[pl.ds(j * 16, 16)] = a16
    pltpu.sync_copy(ag_v, ag_hbm.at[w])


def _sc_alpha(targets):
    t2 = targets.reshape(_NW, 4, 128)
    mesh = plsc.VectorSubcoreMesh(core_axis_name="c", subcore_axis_name="s")
    fn = pl.kernel(
        _sc_alpha_body,
        out_type=[
            jax.ShapeDtypeStruct((_NW, _RW), jnp.float32),
            jax.ShapeDtypeStruct((2, _HC), jnp.float32),
        ],
        mesh=mesh,
        scratch_types=[
            pltpu.VMEM((4, 128), jnp.int32),
            pltpu.VMEM((4, 128), jnp.int32),
            pltpu.VMEM((128,), jnp.float32),
            pltpu.VMEM((_HC,), jnp.float32),
            pltpu.VMEM((_RW,), jnp.float32),
            pltpu.VMEM((_RW,), jnp.float32),
            pltpu.VMEM_SHARED((_HC,), jnp.float32),
        ],
    )
    ag, hist = fn(t2)
    return ag, hist


def _pass_a(x_ref, t_ref, nll_ref):
    i = pl.program_id(0)
    x = x_ref[...]
    r, c = x.shape
    t = t_ref[pl.ds(i, 1), :][0, :]
    lse = jnp.log(jnp.sum(jnp.exp(x), axis=1))
    cols = jax.lax.broadcasted_iota(jnp.int32, (r, c), 1)
    maskf = (cols == t[:, None]).astype(jnp.float32)
    tl = jnp.sum(x * maskf, axis=1)
    nll_ref[0, 0, :] = lse - tl


def _combine(nll_ref, ag_ref, hist_ref, out_ref):
    h = hist_ref[0, :]
    hcols = jax.lax.broadcasted_iota(jnp.int32, (1, _HC), 1)[0, :]
    a = 1.0 / (h * (1.0 / _N) + 1e-5)
    z = jnp.sum(jnp.where(hcols < _C, a, 0.0))
    ce = (ag_ref[...] * (1.0 / z)) * nll_ref[...]
    pt = jnp.exp(-ce)
    om = 1.0 - pt
    out_ref[...] = jnp.broadcast_to(jnp.sum(om * om * ce) * (1.0 / _N), (1, 1))


def kernel(inputs, targets):
    n, c = inputs.shape
    t32 = targets.astype(jnp.int32)

    ag, hist = _sc_alpha(t32)

    ra = 2048
    ga = n // ra
    t2d = t32.reshape(ga, ra)

    nll = pl.pallas_call(
        _pass_a,
        grid=(ga,),
        in_specs=[
            pl.BlockSpec((ra, c), lambda i: (i, 0)),
            pl.BlockSpec((ga, ra), lambda i: (0, 0)),
        ],
        out_specs=pl.BlockSpec((1, 1, ra), lambda i: (i, 0, 0)),
        out_shape=jax.ShapeDtypeStruct((ga, 1, ra), jnp.float32),
    )(inputs, t2d)

    loss = pl.pallas_call(
        _combine,
        in_specs=[
            pl.BlockSpec((128, 128), lambda: (0, 0)),
            pl.BlockSpec((128, 128), lambda: (0, 0)),
            pl.BlockSpec((1, _HC), lambda: (0, 0)),
        ],
        out_specs=pl.BlockSpec((1, 1), lambda: (0, 0)),
        out_shape=jax.ShapeDtypeStruct((1, 1), jnp.float32),
    )(nll.reshape(128, 128), ag.reshape(128, 128), hist[0:1])

    return loss[0, 0]


# speedup vs baseline: 1.9160x; 1.9160x over previous
import functools

import jax
import jax.numpy as jnp
from jax import lax
from jax.experimental import pallas as pl
from jax.experimental.pallas import tpu as pltpu
from jax.experimental.pallas import tpu_sc as plsc

_N = 16384
_C = 1000
_HC = 1024
_NW = 32
_RW = _N // _NW


def _sc_alpha_body(t2_hbm, ag_hbm, hist_hbm, tgth_v, tgt2_v, ones_v,
                   zero_v, htg_v, ag_v, hist_sh):
    c = lax.axis_index("c")
    s = lax.axis_index("s")
    w = s * 2 + c

    for i in range(8):
        ones_v[pl.ds(i * 16, 16)] = jnp.ones((16,), jnp.float32)
    for i in range(64):
        zero_v[pl.ds(i * 16, 16)] = jnp.zeros((16,), jnp.float32)

    pltpu.sync_copy(t2_hbm.at[w], tgt2_v)

    @pl.when(s == 0)
    def _():
        pltpu.sync_copy(zero_v, hist_sh)

    plsc.subcore_barrier()
    for k in range(2):
        pltpu.sync_copy(t2_hbm.at[2 * s + k], tgth_v)
        for q in range(4):
            pltpu.sync_copy(ones_v, hist_sh.at[tgth_v.at[q]], add=True)
    plsc.subcore_barrier()

    @pl.when(s == 0)
    def _():
        pltpu.sync_copy(hist_sh, hist_hbm.at[c])

    plsc.subcore_barrier()

    for q in range(4):
        pltpu.sync_copy(hist_sh.at[tgt2_v.at[q]],
                        htg_v.at[pl.ds(q * 128, 128)])
    inv_n = 1.0 / _N
    for j in range(_RW // 16):
        ht16 = htg_v[pl.ds(j * 16, 16)]
        a16 = 1.0 / (ht16 * inv_n + 1e-5)
        ag_v[pl.ds(j * 16, 16)] = a16
    pltpu.sync_copy(ag_v, ag_hbm.at[w])


def _sc_alpha(targets):
    t2 = targets.reshape(_NW, 4, 128)
    mesh = plsc.VectorSubcoreMesh(core_axis_name="c", subcore_axis_name="s")
    fn = pl.kernel(
        _sc_alpha_body,
        out_type=[
            jax.ShapeDtypeStruct((_NW, _RW), jnp.float32),
            jax.ShapeDtypeStruct((2, _HC), jnp.float32),
        ],
        mesh=mesh,
        scratch_types=[
            pltpu.VMEM((4, 128), jnp.int32),
            pltpu.VMEM((4, 128), jnp.int32),
            pltpu.VMEM((128,), jnp.float32),
            pltpu.VMEM((_HC,), jnp.float32),
            pltpu.VMEM((_RW,), jnp.float32),
            pltpu.VMEM((_RW,), jnp.float32),
            pltpu.VMEM_SHARED((_HC,), jnp.float32),
        ],
    )
    ag, hist = fn(t2)
    return ag, hist


def _pass_a(x_ref, t_ref, ag_ref, hist_ref, out_ref):
    i = pl.program_id(0)
    xt = x_ref[...]
    c, r = xt.shape
    t = t_ref[pl.ds(i, 1), :][0, :]
    lse = jnp.log(jnp.sum(jnp.exp(xt), axis=0))
    rows = jax.lax.broadcasted_iota(jnp.int32, (c, r), 0)
    maskf = (rows == t[None, :]).astype(jnp.float32)
    tl = jnp.sum(xt * maskf, axis=0)
    nll = lse - tl

    h = hist_ref[0, :]
    hcols = jax.lax.broadcasted_iota(jnp.int32, (1, _HC), 1)[0, :]
    a = 1.0 / (h * (1.0 / _N) + 1e-5)
    z = jnp.sum(jnp.where(hcols < _C, a, 0.0))

    ag = ag_ref[pl.ds(i, 1), :][0, :]
    ce = (ag * (1.0 / z)) * nll
    pt = jnp.exp(-ce)
    om = 1.0 - pt
    ps = jnp.broadcast_to(jnp.sum(om * om * ce) * (1.0 / _N), (1, 1))

    @pl.when(i == 0)
    def _():
        out_ref[...] = ps

    @pl.when(i > 0)
    def _():
        out_ref[...] += ps


def kernel(inputs, targets):
    n, c = inputs.shape
    t32 = targets.astype(jnp.int32)

    ag, hist = _sc_alpha(t32)

    ra = 2048
    ga = n // ra
    t2d = t32.reshape(ga, ra)
    ag2 = ag.reshape(ga, ra)
    xt = inputs.T

    loss = pl.pallas_call(
        _pass_a,
        grid=(ga,),
        in_specs=[
            pl.BlockSpec((c, ra), lambda i: (0, i)),
            pl.BlockSpec((ga, ra), lambda i: (0, 0)),
            pl.BlockSpec((ga, ra), lambda i: (0, 0)),
            pl.BlockSpec((1, _HC), lambda i: (0, 0)),
        ],
        out_specs=pl.BlockSpec((1, 1), lambda i: (0, 0)),
        out_shape=jax.ShapeDtypeStruct((1, 1), jnp.float32),
    )(xt, t2d, ag2, hist[0:1])

    return loss[0, 0]


# speedup vs baseline: 2.1924x; 1.1442x over previous
import functools

import jax
import jax.numpy as jnp
from jax import lax
from jax.experimental import pallas as pl
from jax.experimental.pallas import tpu as pltpu
from jax.experimental.pallas import tpu_sc as plsc

_N = 16384
_C = 1000
_HC = 1024
_NW = 32
_RW = _N // _NW


def _sc_alpha_body(t2_hbm, ag_hbm, hist_hbm, tgth_v, tgt2_v, ones_v,
                   zero_v, htg_v, ag_v, hist_sh):
    c = lax.axis_index("c")
    s = lax.axis_index("s")
    w = s * 2 + c

    for i in range(8):
        ones_v[pl.ds(i * 16, 16)] = jnp.ones((16,), jnp.float32)
    for i in range(64):
        zero_v[pl.ds(i * 16, 16)] = jnp.zeros((16,), jnp.float32)

    pltpu.sync_copy(t2_hbm.at[w], tgt2_v)

    @pl.when(s == 0)
    def _():
        pltpu.sync_copy(zero_v, hist_sh)

    plsc.subcore_barrier()
    for k in range(2):
        pltpu.sync_copy(t2_hbm.at[2 * s + k], tgth_v)
        for q in range(4):
            pltpu.sync_copy(ones_v, hist_sh.at[tgth_v.at[q]], add=True)
    plsc.subcore_barrier()

    @pl.when(s == 0)
    def _():
        pltpu.sync_copy(hist_sh, hist_hbm.at[c])

    plsc.subcore_barrier()

    for q in range(4):
        pltpu.sync_copy(hist_sh.at[tgt2_v.at[q]],
                        htg_v.at[pl.ds(q * 128, 128)])
    inv_n = 1.0 / _N
    for j in range(_RW // 16):
        ht16 = htg_v[pl.ds(j * 16, 16)]
        a16 = 1.0 / (ht16 * inv_n + 1e-5)
        ag_v[pl.ds(j * 16, 16)] = a16
    pltpu.sync_copy(ag_v, ag_hbm.at[w])


def _sc_alpha(targets):
    t2 = targets.reshape(_NW, 4, 128)
    mesh = plsc.VectorSubcoreMesh(core_axis_name="c", subcore_axis_name="s")
    fn = pl.kernel(
        _sc_alpha_body,
        out_type=[
            jax.ShapeDtypeStruct((_NW, _RW), jnp.float32),
            jax.ShapeDtypeStruct((2, _HC), jnp.float32),
        ],
        mesh=mesh,
        scratch_types=[
            pltpu.VMEM((4, 128), jnp.int32),
            pltpu.VMEM((4, 128), jnp.int32),
            pltpu.VMEM((128,), jnp.float32),
            pltpu.VMEM((_HC,), jnp.float32),
            pltpu.VMEM((_RW,), jnp.float32),
            pltpu.VMEM((_RW,), jnp.float32),
            pltpu.VMEM_SHARED((_HC,), jnp.float32),
        ],
    )
    ag, hist = fn(t2)
    return ag, hist


def _pass_a(x_ref, t_ref, nll_ref):
    i = pl.program_id(0)
    xt = x_ref[...]
    c, r = xt.shape
    t = t_ref[pl.ds(i, 1), :][0, :]
    lse = jnp.log(jnp.sum(jnp.exp(xt), axis=0))
    rows = jax.lax.broadcasted_iota(jnp.int32, (c, r), 0)
    maskf = (rows == t[None, :]).astype(jnp.float32)
    tl = jnp.sum(xt * maskf, axis=0)
    nll_ref[0, 0, :] = lse - tl


def _combine(nll_ref, ag_ref, hist_ref, out_ref):
    h = hist_ref[0, :]
    hcols = jax.lax.broadcasted_iota(jnp.int32, (1, _HC), 1)[0, :]
    a = 1.0 / (h * (1.0 / _N) + 1e-5)
    z = jnp.sum(jnp.where(hcols < _C, a, 0.0))
    ce = (ag_ref[...] * (1.0 / z)) * nll_ref[...]
    pt = jnp.exp(-ce)
    om = 1.0 - pt
    out_ref[...] = jnp.broadcast_to(jnp.sum(om * om * ce) * (1.0 / _N), (1, 1))


def kernel(inputs, targets):
    n, c = inputs.shape
    t32 = targets.astype(jnp.int32)

    ag, hist = _sc_alpha(t32)

    ra = 2048
    ga = n // ra
    t2d = t32.reshape(ga, ra)
    xt = inputs.T

    nll = pl.pallas_call(
        _pass_a,
        grid=(ga,),
        in_specs=[
            pl.BlockSpec((c, ra), lambda i: (0, i)),
            pl.BlockSpec((ga, ra), lambda i: (0, 0)),
        ],
        out_specs=pl.BlockSpec((1, 1, ra), lambda i: (i, 0, 0)),
        out_shape=jax.ShapeDtypeStruct((ga, 1, ra), jnp.float32),
    )(xt, t2d)

    loss = pl.pallas_call(
        _combine,
        in_specs=[
            pl.BlockSpec((128, 128), lambda: (0, 0)),
            pl.BlockSpec((128, 128), lambda: (0, 0)),
            pl.BlockSpec((1, _HC), lambda: (0, 0)),
        ],
        out_specs=pl.BlockSpec((1, 1), lambda: (0, 0)),
        out_shape=jax.ShapeDtypeStruct((1, 1), jnp.float32),
    )(nll.reshape(128, 128), ag.reshape(128, 128), hist[0:1])

    return loss[0, 0]
